# Initial kernel scaffold; baseline (speedup 1.0000x reference)
import functools

import jax
import jax.numpy as jnp
from jax import lax
from jax.experimental import pallas as pl
from jax.experimental.pallas import tpu as pltpu
from jax.experimental.pallas import tpu_sc as plsc

NC = 2
NS = 16
NW = NC * NS
CH = 128
NBUF = 8
DEPTH = 4


@functools.lru_cache(maxsize=None)
def _make_gather(V, D, nchunk):
    mesh = plsc.VectorSubcoreMesh(core_axis_name="c", subcore_axis_name="s")

    @functools.partial(
        pl.kernel,
        mesh=mesh,
        out_type=jax.ShapeDtypeStruct((NW, nchunk, CH, D), jnp.float32),
        scratch_types=[
            pltpu.VMEM((nchunk, CH), jnp.int32),
            pltpu.VMEM((NBUF, CH, D), jnp.float32),
            *(pltpu.SemaphoreType.DMA for _ in range(2 * NBUF)),
        ],
    )
    def k(table_hbm, idx_hbm, out_hbm, idx_v, rows_v, *sems):
        gsem = sems[:NBUF]
        wsem = sems[NBUF:]
        wid = lax.axis_index("s") * NC + lax.axis_index("c")

        pltpu.sync_copy(idx_hbm.at[wid], idx_v)

        def g_copy(j, s):
            return pltpu.make_async_copy(
                table_hbm.at[idx_v.at[j]], rows_v.at[s], gsem[s])

        def w_copy(j, s):
            return pltpu.make_async_copy(
                rows_v.at[s], out_hbm.at[wid, j], wsem[s])

        for b in range(DEPTH):
            g_copy(b, b).start()

        def body(i, carry):
            for b in range(NBUF):
                j = i * NBUF + b
                g_copy(j, b).wait()
                w_copy(j, b).start()
                t = (b + DEPTH) % NBUF

                @pl.when(j >= DEPTH)
                def _():
                    w_copy(j - DEPTH, t).wait()

                @pl.when(j + DEPTH < nchunk)
                def _():
                    g_copy(j + DEPTH, t).start()
            return carry

        lax.fori_loop(0, nchunk // NBUF, body, 0)

        for b in range(DEPTH):
            w_copy(nchunk - DEPTH + b, DEPTH + b).wait()

    return k


def kernel(x, table):
    B0, H = x.shape
    V, D = table.shape
    B = B0 * H
    idx = x.reshape(B).astype(jnp.int32)

    per_w = -(-B // (NW * CH)) * CH
    nchunk = per_w // CH
    if nchunk % NBUF:
        nchunk += NBUF - nchunk % NBUF
    Bpad = NW * nchunk * CH
    if Bpad != B:
        idx = jnp.pad(idx, (0, Bpad - B))

    out = _make_gather(V, D, nchunk)(table, idx.reshape(NW, nchunk, CH))
    out = out.reshape(Bpad, D)
    if Bpad != B:
        out = out[:B]
    return out.reshape(B0, H, D)


# speedup vs baseline: 1.8771x; 1.8771x over previous
import functools

import jax
import jax.numpy as jnp
from jax import lax
from jax.experimental import pallas as pl
from jax.experimental.pallas import tpu as pltpu
from jax.experimental.pallas import tpu_sc as plsc

NC = 2
NS = 16
NW = NC * NS
CH = 128
NBUF = 8
DEPTH = 4


@functools.lru_cache(maxsize=None)
def _make_gather(V, D, nchunk):
    mesh = plsc.VectorSubcoreMesh(core_axis_name="c", subcore_axis_name="s")

    @functools.partial(
        pl.kernel,
        mesh=mesh,
        out_type=jax.ShapeDtypeStruct((NW, nchunk, CH, D), jnp.float32),
        scratch_types=[
            pltpu.VMEM((nchunk, CH), jnp.int32),
            pltpu.VMEM((NBUF, CH, D), jnp.float32),
            *(pltpu.SemaphoreType.DMA for _ in range(2 * NBUF)),
        ],
        compiler_params=pltpu.CompilerParams(use_tc_tiling_on_sc=False),
    )
    def k(table_hbm, idx_hbm, out_hbm, idx_v, rows_v, *sems):
        gsem = sems[:NBUF]
        wsem = sems[NBUF:]
        wid = lax.axis_index("s") * NC + lax.axis_index("c")

        pltpu.sync_copy(idx_hbm.at[wid], idx_v)

        def g_copy(j, s):
            return pltpu.make_async_copy(
                table_hbm.at[idx_v.at[j]], rows_v.at[s], gsem[s])

        def w_copy(j, s):
            return pltpu.make_async_copy(
                rows_v.at[s], out_hbm.at[wid, j], wsem[s])

        for b in range(DEPTH):
            g_copy(b, b).start()

        def body(i, carry):
            for b in range(NBUF):
                j = i * NBUF + b
                g_copy(j, b).wait()
                w_copy(j, b).start()
                t = (b + DEPTH) % NBUF

                @pl.when(j >= DEPTH)
                def _():
                    w_copy(j - DEPTH, t).wait()

                @pl.when(j + DEPTH < nchunk)
                def _():
                    g_copy(j + DEPTH, t).start()
            return carry

        lax.fori_loop(0, nchunk // NBUF, body, 0)

        for b in range(DEPTH):
            w_copy(nchunk - DEPTH + b, DEPTH + b).wait()

    return k


def kernel(x, table):
    B0, H = x.shape
    V, D = table.shape
    B = B0 * H
    idx = x.reshape(B).astype(jnp.int32)

    per_w = -(-B // (NW * CH)) * CH
    nchunk = per_w // CH
    if nchunk % NBUF:
        nchunk += NBUF - nchunk % NBUF
    Bpad = NW * nchunk * CH
    if Bpad != B:
        idx = jnp.pad(idx, (0, Bpad - B))

    out = _make_gather(V, D, nchunk)(table, idx.reshape(NW, nchunk, CH))
    out = out.reshape(Bpad, D)
    if Bpad != B:
        out = out[:B]
    return out.reshape(B0, H, D)
